# Initial kernel scaffold; baseline (speedup 1.0000x reference)
import jax
import jax.numpy as jnp
from jax.experimental import pallas as pl


def kernel(x):
    raise NotImplementedError("write your pallas kernel here")


# speedup vs baseline: 42.6257x; 42.6257x over previous
import jax
import jax.numpy as jnp
from jax import lax
from jax.experimental import pallas as pl
from jax.experimental.pallas import tpu as pltpu
from jax.experimental.pallas import tpu_sc as plsc

_B, _C, _W, _H = 4, 96, 224, 224
_N = _W * _H
_GROUPS, _G = 12, 8
_NWORKERS = 32
_SPB = _NWORKERS // _B
_SLAB = _N // _SPB
_CHUNK = 896
_NCHUNK = _SLAB // _CHUNK
_LANES = 16
_JN = _CHUNK // _LANES


def _body(x_hbm, out_hbm, buf):
    cid = lax.axis_index("c")
    sid = lax.axis_index("s")
    wid = sid * 2 + cid
    b = wid // _SPB
    s = wid % _SPB

    def chunk_step(k, carry):
        p0 = s * _SLAB + k * _CHUNK
        pltpu.sync_copy(x_hbm.at[b, :, pl.ds(p0, _CHUNK)], buf)

        def group_step(g, carry2):
            def vec_step(j, carry3):
                col = j * _LANES
                v = [buf[g * _G + i, pl.ds(col, _LANES)] for i in range(_G)]
                m = v[0]
                for i in range(1, _G):
                    m = jnp.maximum(m, v[i])
                r = jnp.maximum(m, 0.0)
                zero = jnp.zeros((_LANES,), jnp.float32)
                pmax = jnp.full((_LANES,), -jnp.inf, jnp.float32)
                for i in range(_G):
                    sel = jnp.logical_and(v[i] == m, pmax < m)
                    if i < _G - 1:
                        pmax = jnp.maximum(pmax, v[i])
                    buf[g * _G + i, pl.ds(col, _LANES)] = jnp.where(sel, r, zero)
                return carry3

            return lax.fori_loop(0, _JN, vec_step, carry2)

        lax.fori_loop(0, _GROUPS, group_step, 0)
        pltpu.sync_copy(buf, out_hbm.at[b, :, pl.ds(p0, _CHUNK)])
        return carry

    lax.fori_loop(0, _NCHUNK, chunk_step, 0)


def kernel(x):
    assert x.shape == (_B, _C, _W, _H) and x.dtype == jnp.float32
    x3 = x.reshape(_B, _C, _N)
    mesh = plsc.VectorSubcoreMesh(core_axis_name="c", subcore_axis_name="s")
    out3 = pl.kernel(
        _body,
        out_type=jax.ShapeDtypeStruct((_B, _C, _N), jnp.float32),
        mesh=mesh,
        scratch_types=[pltpu.VMEM((_C, _CHUNK), jnp.float32)],
    )(x3)
    return out3.reshape(_B, _C, _W, _H)


# speedup vs baseline: 76.8076x; 1.8019x over previous
import jax
import jax.numpy as jnp
from jax import lax
from jax.experimental import pallas as pl
from jax.experimental.pallas import tpu as pltpu
from jax.experimental.pallas import tpu_sc as plsc

_B, _C, _W, _H = 4, 96, 224, 224
_G = 8
_CH = 48
_GPU = _CH // _G
_WT = 8
_NWT = _W // _WT
_UNITS = _B * (_C // _CH) * _NWT
_NWORKERS = 32
_UPW = _UNITS // _NWORKERS
_LANES = 16
_JH = _H // _LANES


def _body(x_hbm, out_hbm, buf):
    cid = lax.axis_index("c")
    sid = lax.axis_index("s")
    wid = sid * 2 + cid

    def unit_step(u, carry):
        uid = wid * _UPW + u
        b = uid // (2 * _NWT)
        rem = uid % (2 * _NWT)
        c0 = (rem // _NWT) * _CH
        w0 = (rem % _NWT) * _WT
        pltpu.sync_copy(
            x_hbm.at[b, pl.ds(c0, _CH), pl.ds(w0, _WT), :], buf)

        def group_step(g, carry2):
            def row_step(w, carry3):
                def vec_step(j, carry4):
                    col = j * _LANES
                    v = [buf[g * _G + i, w, pl.ds(col, _LANES)]
                         for i in range(_G)]
                    m = v[0]
                    for i in range(1, _G):
                        m = jnp.maximum(m, v[i])
                    r = jnp.maximum(m, 0.0)
                    zero = jnp.zeros((_LANES,), jnp.float32)
                    pmax = jnp.full((_LANES,), -jnp.inf, jnp.float32)
                    for i in range(_G):
                        sel = jnp.logical_and(v[i] == m, pmax < m)
                        if i < _G - 1:
                            pmax = jnp.maximum(pmax, v[i])
                        buf[g * _G + i, w, pl.ds(col, _LANES)] = (
                            jnp.where(sel, r, zero))
                    return carry4

                return lax.fori_loop(0, _JH, vec_step, carry3)

            return lax.fori_loop(0, _WT, row_step, carry2)

        lax.fori_loop(0, _GPU, group_step, 0)
        pltpu.sync_copy(
            buf, out_hbm.at[b, pl.ds(c0, _CH), pl.ds(w0, _WT), :])
        return carry

    lax.fori_loop(0, _UPW, unit_step, 0)


def kernel(x):
    assert x.shape == (_B, _C, _W, _H) and x.dtype == jnp.float32
    mesh = plsc.VectorSubcoreMesh(core_axis_name="c", subcore_axis_name="s")
    return pl.kernel(
        _body,
        out_type=jax.ShapeDtypeStruct((_B, _C, _W, _H), jnp.float32),
        mesh=mesh,
        scratch_types=[pltpu.VMEM((_CH, _WT, _H), jnp.float32)],
    )(x)


# speedup vs baseline: 97.5495x; 1.2701x over previous
import jax
import jax.numpy as jnp
from jax import lax
from jax.experimental import pallas as pl
from jax.experimental.pallas import tpu as pltpu
from jax.experimental.pallas import tpu_sc as plsc

_B, _C, _W, _H = 4, 96, 224, 224
_G = 8
_CH = 24
_GPU = _CH // _G
_WT = 8
_NWT = _W // _WT
_UNITS = _B * (_C // _CH) * _NWT
_NWORKERS = 32
_UPW = _UNITS // _NWORKERS
_LANES = 16
_JH = _H // _LANES


def _body(x_hbm, out_hbm, buf0, buf1, si0, si1, so0, so1):
    cid = lax.axis_index("c")
    sid = lax.axis_index("s")
    wid = sid * 2 + cid
    bufs, sins, souts = (buf0, buf1), (si0, si1), (so0, so1)

    def slices(u):
        uid = wid * _UPW + u
        b = uid // ((_C // _CH) * _NWT)
        rem = uid % ((_C // _CH) * _NWT)
        c0 = (rem // _NWT) * _CH
        w0 = (rem % _NWT) * _WT
        return b, c0, w0

    def start_in(u, p):
        b, c0, w0 = slices(u)
        return pltpu.async_copy(
            x_hbm.at[b, pl.ds(c0, _CH), pl.ds(w0, _WT), :], bufs[p], sins[p])

    def start_out(u, p):
        b, c0, w0 = slices(u)
        return pltpu.async_copy(
            bufs[p], out_hbm.at[b, pl.ds(c0, _CH), pl.ds(w0, _WT), :],
            souts[p])

    def compute(p):
        buf = bufs[p]

        def group_step(g, carry2):
            def row_step(w, carry3):
                def vec_step(j, carry4):
                    col = j * _LANES
                    v = [buf[g * _G + i, w, pl.ds(col, _LANES)]
                         for i in range(_G)]
                    m = v[0]
                    for i in range(1, _G):
                        m = jnp.maximum(m, v[i])
                    r = jnp.maximum(m, 0.0)
                    zero = jnp.zeros((_LANES,), jnp.float32)
                    pmax = jnp.full((_LANES,), -jnp.inf, jnp.float32)
                    for i in range(_G):
                        sel = jnp.logical_and(v[i] == m, pmax < m)
                        if i < _G - 1:
                            pmax = jnp.maximum(pmax, v[i])
                        buf[g * _G + i, w, pl.ds(col, _LANES)] = (
                            jnp.where(sel, r, zero))
                    return carry4

                return lax.fori_loop(0, _JH, vec_step, carry3)

            return lax.fori_loop(0, _WT, row_step, carry2)

        lax.fori_loop(0, _GPU, group_step, 0)

    in_h = [None] * _UPW
    out_h = [None] * _UPW
    in_h[0] = start_in(0, 0)
    for u in range(_UPW):
        p = u % 2
        in_h[u].wait()
        if u + 1 < _UPW:
            if u >= 1:
                out_h[u - 1].wait()
            in_h[u + 1] = start_in(u + 1, 1 - p)
        compute(p)
        out_h[u] = start_out(u, p)
    out_h[_UPW - 2].wait()
    out_h[_UPW - 1].wait()


def kernel(x):
    assert x.shape == (_B, _C, _W, _H) and x.dtype == jnp.float32
    mesh = plsc.VectorSubcoreMesh(core_axis_name="c", subcore_axis_name="s")
    return pl.kernel(
        _body,
        out_type=jax.ShapeDtypeStruct((_B, _C, _W, _H), jnp.float32),
        mesh=mesh,
        scratch_types=[
            pltpu.VMEM((_CH, _WT, _H), jnp.float32),
            pltpu.VMEM((_CH, _WT, _H), jnp.float32),
            pltpu.SemaphoreType.DMA,
            pltpu.SemaphoreType.DMA,
            pltpu.SemaphoreType.DMA,
            pltpu.SemaphoreType.DMA,
        ],
    )(x)


# speedup vs baseline: 129.4365x; 1.3269x over previous
import jax
import jax.numpy as jnp
from jax import lax
from jax.experimental import pallas as pl
from jax.experimental.pallas import tpu as pltpu
from jax.experimental.pallas import tpu_sc as plsc

_B, _C, _W, _H = 4, 96, 224, 224
_G = 8
_CH = 24
_GPU = _CH // _G
_WT = 8
_NWT = _W // _WT
_UNITS = _B * (_C // _CH) * _NWT
_NWORKERS = 32
_UPW = _UNITS // _NWORKERS
_LANES = 16
_JH = _H // _LANES


def _body(x_hbm, out_hbm, buf0, buf1, si0, si1, so0, so1):
    cid = lax.axis_index("c")
    sid = lax.axis_index("s")
    wid = sid * 2 + cid
    bufs, sins, souts = (buf0, buf1), (si0, si1), (so0, so1)

    def slices(u):
        uid = wid * _UPW + u
        b = uid // ((_C // _CH) * _NWT)
        rem = uid % ((_C // _CH) * _NWT)
        c0 = (rem // _NWT) * _CH
        w0 = (rem % _NWT) * _WT
        return b, c0, w0

    def start_in(u, p):
        b, c0, w0 = slices(u)
        return pltpu.async_copy(
            x_hbm.at[b, pl.ds(c0, _CH), pl.ds(w0, _WT), :], bufs[p], sins[p])

    def start_out(u, p):
        b, c0, w0 = slices(u)
        return pltpu.async_copy(
            bufs[p], out_hbm.at[b, pl.ds(c0, _CH), pl.ds(w0, _WT), :],
            souts[p])

    def compute(p):
        buf = bufs[p]

        @plsc.parallel_loop(0, _GPU * _WT * _JH, unroll=2)
        def _vec_step(t):
            g = t // (_WT * _JH)
            rem = t % (_WT * _JH)
            w = rem // _JH
            col = (rem % _JH) * _LANES
            v = [buf[g * _G + i, w, pl.ds(col, _LANES)] for i in range(_G)]
            m = v[0]
            for i in range(1, _G):
                m = jnp.maximum(m, v[i])
            r = jnp.maximum(m, 0.0)
            zero = jnp.zeros((_LANES,), jnp.float32)
            pmax = jnp.full((_LANES,), -jnp.inf, jnp.float32)
            for i in range(_G):
                sel = jnp.logical_and(v[i] == m, pmax < m)
                if i < _G - 1:
                    pmax = jnp.maximum(pmax, v[i])
                buf[g * _G + i, w, pl.ds(col, _LANES)] = (
                    jnp.where(sel, r, zero))

    in_h = [None] * _UPW
    out_h = [None] * _UPW
    in_h[0] = start_in(0, 0)
    for u in range(_UPW):
        p = u % 2
        in_h[u].wait()
        if u + 1 < _UPW:
            if u >= 1:
                out_h[u - 1].wait()
            in_h[u + 1] = start_in(u + 1, 1 - p)
        compute(p)
        out_h[u] = start_out(u, p)
    out_h[_UPW - 2].wait()
    out_h[_UPW - 1].wait()


def kernel(x):
    assert x.shape == (_B, _C, _W, _H) and x.dtype == jnp.float32
    mesh = plsc.VectorSubcoreMesh(core_axis_name="c", subcore_axis_name="s")
    return pl.kernel(
        _body,
        out_type=jax.ShapeDtypeStruct((_B, _C, _W, _H), jnp.float32),
        mesh=mesh,
        scratch_types=[
            pltpu.VMEM((_CH, _WT, _H), jnp.float32),
            pltpu.VMEM((_CH, _WT, _H), jnp.float32),
            pltpu.SemaphoreType.DMA,
            pltpu.SemaphoreType.DMA,
            pltpu.SemaphoreType.DMA,
            pltpu.SemaphoreType.DMA,
        ],
    )(x)
